# baseline (speedup 1.0000x reference)
import jax
import jax.numpy as jnp
from jax.experimental import pallas as pl

_NB = 1024
_G = 64


def _gcn_body(xp_ref, acc_ref, dinv_ref, invdeg_ref, wmat_ref,
              w1_ref, b1_ref, w2_ref, b2_ref, w3_ref, b3_ref, invcnt_ref,
              p_ref, out_ref):
    i = pl.program_id(0)
    dinv = dinv_ref[...]
    s1 = acc_ref[...] + invdeg_ref[...] * xp_ref[...]
    h1 = jnp.maximum(
        jnp.dot(s1, w1_ref[...], preferred_element_type=jnp.float32)
        + b1_ref[0:1, :], 0.0)
    g2 = dinv * jnp.dot(h1, w2_ref[...],
                        preferred_element_type=jnp.float32)
    part = jnp.dot(wmat_ref[...], g2,
                   preferred_element_type=jnp.float32)

    @pl.when(i == 0)
    def _init():
        p_ref[...] = jnp.zeros_like(p_ref)

    p_ref[...] += part

    @pl.when(i == pl.num_programs(0) - 1)
    def _final():
        c = out_ref.shape[1]
        pooled = p_ref[...] * invcnt_ref[...] + b2_ref[0:1, :]
        full = jnp.dot(pooled, w3_ref[...],
                       preferred_element_type=jnp.float32)
        out_ref[...] = full[:, :c] + b3_ref[0:1, :c]


def kernel(x, edge_index, batch, W1, b1, W2, b2, W3, b3):
    n, f_in = x.shape
    h = W2.shape[1]
    c = W3.shape[1]
    g = _G
    nblk = (n + _NB - 1) // _NB
    npad = nblk * _NB

    src = edge_index[0]
    dst = edge_index[1]
    x = x.astype(jnp.float32)

    deg = jnp.ones((n,), jnp.float32).at[dst].add(1.0)
    dinv = jax.lax.rsqrt(deg)
    invdeg = 1.0 / deg

    xp = jnp.zeros((npad, 16), jnp.float32).at[:n, :f_in].set(x)
    norm = dinv[src] * dinv[dst]
    acc = jnp.zeros((npad, 16), jnp.float32).at[dst].add(
        xp[src] * norm[:, None])

    wmat = (jnp.zeros((g, npad), jnp.float32)
            .at[batch[dst], src].add(dinv[dst])
            .at[batch, jnp.arange(n)].add(dinv))

    cnt = jnp.zeros((g,), jnp.float32).at[batch].add(1.0)
    invcnt = (1.0 / jnp.clip(cnt, 1.0)).reshape(g, 1)

    dinv_p = jnp.zeros((npad, 1), jnp.float32).at[:n, 0].set(dinv)
    invdeg_p = jnp.zeros((npad, 1), jnp.float32).at[:n, 0].set(invdeg)

    w1p = jnp.zeros((16, h), jnp.float32).at[:f_in].set(W1)
    b1p = jnp.zeros((8, h), jnp.float32).at[0].set(b1)
    b2p = jnp.zeros((8, h), jnp.float32).at[0].set(b2)
    w3p = jnp.zeros((h, 128), jnp.float32).at[:, :c].set(W3)
    b3p = jnp.zeros((8, 128), jnp.float32).at[0, :c].set(b3)

    blk = lambda i: (i, 0)
    fixed = lambda i: (0, 0)
    _, out = pl.pallas_call(
        _gcn_body,
        grid=(nblk,),
        in_specs=[
            pl.BlockSpec((_NB, 16), blk),
            pl.BlockSpec((_NB, 16), blk),
            pl.BlockSpec((_NB, 1), blk),
            pl.BlockSpec((_NB, 1), blk),
            pl.BlockSpec((g, _NB), lambda i: (0, i)),
            pl.BlockSpec((16, h), fixed),
            pl.BlockSpec((8, h), fixed),
            pl.BlockSpec((h, h), fixed),
            pl.BlockSpec((8, h), fixed),
            pl.BlockSpec((h, 128), fixed),
            pl.BlockSpec((8, 128), fixed),
            pl.BlockSpec((g, 1), fixed),
        ],
        out_specs=[
            pl.BlockSpec((g, h), fixed),
            pl.BlockSpec((g, c), fixed),
        ],
        out_shape=[
            jax.ShapeDtypeStruct((g, h), jnp.float32),
            jax.ShapeDtypeStruct((g, c), jnp.float32),
        ],
    )(xp, acc, dinv_p, invdeg_p, wmat, w1p, b1p, W2, b2p, w3p, b3p, invcnt)
    return out
